# baseline (speedup 1.0000x reference)
import jax
import jax.numpy as jnp
from jax import lax
from jax.experimental import pallas as pl
from jax.experimental.pallas import tpu as pltpu
from jax.experimental.pallas import tpu_sc as plsc

B = 4096
N = 50
NP = 56
D = 64
D2 = 2 * D
NC = 2
NS = 16
NW = NC * NS

BW = B // NW
CHUNK = 128
HOP_ROWS = B * NP
PER_W = HOP_ROWS // NW
N_CHUNKS = PER_W // CHUNK


def _sc_gather_body(combo, i1idx, uidx, i3idx, tgt,
                    g1, g2, g3, gt, idx_v, tidx_v, rows_v, sem, wsem):
    wid = lax.axis_index("s") * NC + lax.axis_index("c")

    def do_hop(idx_hbm, out_hbm):
        pltpu.sync_copy(idx_hbm.at[wid], idx_v)

        def step(c, carry):
            pltpu.async_copy(combo.at[idx_v.at[c]], rows_v, sem).wait()
            pltpu.async_copy(
                rows_v, out_hbm.at[pl.ds(wid * PER_W + c * CHUNK, CHUNK)],
                wsem).wait()
            return carry

        lax.fori_loop(0, N_CHUNKS, step, 0)

    do_hop(i1idx, g1)
    do_hop(uidx, g2)
    do_hop(i3idx, g3)

    pltpu.sync_copy(tgt.at[pl.ds(wid * BW, BW)], tidx_v)
    pltpu.async_copy(combo.at[tidx_v], rows_v.at[pl.ds(0, BW)], sem).wait()
    pltpu.async_copy(rows_v.at[pl.ds(0, BW)], gt.at[pl.ds(wid * BW, BW)],
                     wsem).wait()


def _make_sc_gather():
    mesh = plsc.VectorSubcoreMesh(core_axis_name="c", subcore_axis_name="s")
    return pl.kernel(
        _sc_gather_body,
        out_type=(
            jax.ShapeDtypeStruct((HOP_ROWS, D2), jnp.float32),
            jax.ShapeDtypeStruct((HOP_ROWS, D2), jnp.float32),
            jax.ShapeDtypeStruct((HOP_ROWS, D2), jnp.float32),
            jax.ShapeDtypeStruct((B, D2), jnp.float32),
        ),
        mesh=mesh,
        scratch_types=[
            pltpu.VMEM((N_CHUNKS, CHUNK), jnp.int32),
            pltpu.VMEM((BW,), jnp.int32),
            pltpu.VMEM((CHUNK, D2), jnp.float32),
            pltpu.SemaphoreType.DMA,
            pltpu.SemaphoreType.DMA,
        ],
        compiler_params=pltpu.CompilerParams(use_tc_tiling_on_sc=True),
    )


BB = 128


def _gat_block(embf, aw_h, av):
    t = jnp.tanh(jnp.dot(embf, aw_h, preferred_element_type=jnp.float32))
    s = jnp.dot(t, av, preferred_element_type=jnp.float32)
    s3 = s.reshape(BB, NP, 1)
    nidx = lax.broadcasted_iota(jnp.int32, (BB, NP, 1), 1)
    s3 = jnp.where(nidx < N, s3, -1e30)
    m = jnp.max(s3, axis=1, keepdims=True)
    e = jnp.exp(s3 - m)
    alpha = e / jnp.sum(e, axis=1, keepdims=True)
    w = embf * alpha.reshape(BB * NP, 1)
    return jnp.sum(w.reshape(BB, NP, D2), axis=1)


def _tc_gat_body(g1_ref, g2_ref, g3_ref, gt_ref,
                 awi_ref, awu_ref, av_ref, rwp_ref, out_ref):
    awi = awi_ref[...]
    awu = awu_ref[...]
    av = av_ref[...]
    rwp = rwp_ref[...]

    agg1 = _gat_block(g1_ref[...], awi, av)
    agg2 = _gat_block(g2_ref[...], awu, av)
    agg3 = _gat_block(g3_ref[...], awi, av)
    tgt = gt_ref[...]

    acc = (jnp.dot(agg1, rwp[0], preferred_element_type=jnp.float32)
           + jnp.dot(agg2, rwp[1], preferred_element_type=jnp.float32)
           + jnp.dot(agg3, rwp[2], preferred_element_type=jnp.float32)
           + jnp.dot(tgt, rwp[3], preferred_element_type=jnp.float32))
    out_ref[...] = jnp.tanh(acc)


def _tc_gat(g1, g2, g3, gt, awi, awu, av_col, rwp):
    hop_spec = pl.BlockSpec((BB * NP, D2), lambda i: (i, 0))
    return pl.pallas_call(
        _tc_gat_body,
        grid=(B // BB,),
        in_specs=[
            hop_spec, hop_spec, hop_spec,
            pl.BlockSpec((BB, D2), lambda i: (i, 0)),
            pl.BlockSpec((D2, D), lambda i: (0, 0)),
            pl.BlockSpec((D2, D), lambda i: (0, 0)),
            pl.BlockSpec((D, 1), lambda i: (0, 0)),
            pl.BlockSpec((4, D2, D), lambda i: (0, 0, 0)),
        ],
        out_specs=pl.BlockSpec((BB, D), lambda i: (i, 0)),
        out_shape=jax.ShapeDtypeStruct((B, D), jnp.float32),
    )(g1, g2, g3, gt, awi, awu, av_col, rwp)


def kernel(target_ids, support_1st, support_2nd, support_3rd,
           user_emb, item_emb, att_w, att_v, refine_w):
    combo = jnp.concatenate([item_emb.T, user_emb.T], axis=0).T

    def pad_idx(s):
        return jnp.concatenate([s, s[:, :NP - N]], axis=1).reshape(
            NW, N_CHUNKS, CHUNK)

    i1idx = pad_idx(support_1st)
    uidx = pad_idx(support_2nd)
    i3idx = pad_idx(support_3rd)

    g1, g2, g3, gt = _make_sc_gather()(combo, i1idx, uidx, i3idx, target_ids)

    z = jnp.zeros((D, D), jnp.float32)
    awi = jnp.concatenate([att_w, z], axis=0)
    awu = jnp.concatenate([z, att_w], axis=0)
    rwp = jnp.stack([
        jnp.concatenate([refine_w[0:D], z], axis=0),
        jnp.concatenate([z, refine_w[D:2 * D]], axis=0),
        jnp.concatenate([refine_w[2 * D:3 * D], z], axis=0),
        jnp.concatenate([z, refine_w[3 * D:4 * D]], axis=0),
    ])
    return _tc_gat(g1, g2, g3, gt, awi, awu, att_v.reshape(D, 1), rwp)


# speedup vs baseline: 1.0501x; 1.0501x over previous
import jax
import jax.numpy as jnp
from jax import lax
from jax.experimental import pallas as pl
from jax.experimental.pallas import tpu as pltpu
from jax.experimental.pallas import tpu_sc as plsc

B = 4096
N = 50
NP = 56
D = 64
D2 = 2 * D
NC = 2
NS = 16
NW = NC * NS

BW = B // NW
CHUNK = 128
HOP_ROWS = B * NP
PER_W = HOP_ROWS // NW
N_CHUNKS = PER_W // CHUNK


def _hop_loop(combo, idx_hbm, out_hbm, wid, idx_v, rows_v, sem, wsem):
    pltpu.sync_copy(idx_hbm.at[wid], idx_v)

    def step(c, carry):
        pltpu.async_copy(combo.at[idx_v.at[c]], rows_v, sem).wait()
        pltpu.async_copy(
            rows_v, out_hbm.at[pl.ds(wid * PER_W + c * CHUNK, CHUNK)],
            wsem).wait()
        return carry

    lax.fori_loop(0, N_CHUNKS, step, 0)


def _sc_hop_body(combo, idx, g, idx_v, rows_v, sem, wsem):
    wid = lax.axis_index("s") * NC + lax.axis_index("c")
    _hop_loop(combo, idx, g, wid, idx_v, rows_v, sem, wsem)


def _sc_hop_tgt_body(combo, idx, tgt, g, gt, idx_v, tidx_v, rows_v, sem, wsem):
    wid = lax.axis_index("s") * NC + lax.axis_index("c")
    _hop_loop(combo, idx, g, wid, idx_v, rows_v, sem, wsem)
    pltpu.sync_copy(tgt.at[pl.ds(wid * BW, BW)], tidx_v)
    pltpu.async_copy(combo.at[tidx_v], rows_v.at[pl.ds(0, BW)], sem).wait()
    pltpu.async_copy(rows_v.at[pl.ds(0, BW)], gt.at[pl.ds(wid * BW, BW)],
                     wsem).wait()


_MESH = dict(core_axis_name="c", subcore_axis_name="s")
_CP = pltpu.CompilerParams(use_tc_tiling_on_sc=True)


def _make_sc_hop():
    return pl.kernel(
        _sc_hop_body,
        out_type=jax.ShapeDtypeStruct((HOP_ROWS, D2), jnp.float32),
        mesh=plsc.VectorSubcoreMesh(**_MESH),
        scratch_types=[
            pltpu.VMEM((N_CHUNKS, CHUNK), jnp.int32),
            pltpu.VMEM((CHUNK, D2), jnp.float32),
            pltpu.SemaphoreType.DMA,
            pltpu.SemaphoreType.DMA,
        ],
        compiler_params=_CP,
    )


def _make_sc_hop_tgt():
    return pl.kernel(
        _sc_hop_tgt_body,
        out_type=(
            jax.ShapeDtypeStruct((HOP_ROWS, D2), jnp.float32),
            jax.ShapeDtypeStruct((B, D2), jnp.float32),
        ),
        mesh=plsc.VectorSubcoreMesh(**_MESH),
        scratch_types=[
            pltpu.VMEM((N_CHUNKS, CHUNK), jnp.int32),
            pltpu.VMEM((BW,), jnp.int32),
            pltpu.VMEM((CHUNK, D2), jnp.float32),
            pltpu.SemaphoreType.DMA,
            pltpu.SemaphoreType.DMA,
        ],
        compiler_params=_CP,
    )


BB = 128


def _gat_block(embf, aw_h, av):
    t = jnp.tanh(jnp.dot(embf, aw_h, preferred_element_type=jnp.float32))
    s = jnp.dot(t, av, preferred_element_type=jnp.float32)
    s3 = s.reshape(BB, NP, 1)
    nidx = lax.broadcasted_iota(jnp.int32, (BB, NP, 1), 1)
    s3 = jnp.where(nidx < N, s3, -1e30)
    m = jnp.max(s3, axis=1, keepdims=True)
    e = jnp.exp(s3 - m)
    alpha = e / jnp.sum(e, axis=1, keepdims=True)
    w = embf * alpha.reshape(BB * NP, 1)
    return jnp.sum(w.reshape(BB, NP, D2), axis=1)


def _tc_hop1_body(g_ref, aw_ref, av_ref, rw_ref, p_ref):
    agg = _gat_block(g_ref[...], aw_ref[...], av_ref[...])
    p_ref[...] = jnp.dot(agg, rw_ref[...], preferred_element_type=jnp.float32)


def _tc_hop2_body(g_ref, gt_ref, p1_ref, aw_ref, av_ref, rw_ref, rwt_ref, p_ref):
    agg = _gat_block(g_ref[...], aw_ref[...], av_ref[...])
    p_ref[...] = (jnp.dot(agg, rw_ref[...], preferred_element_type=jnp.float32)
                  + jnp.dot(gt_ref[...], rwt_ref[...],
                            preferred_element_type=jnp.float32)
                  + p1_ref[...])


def _tc_hop3_body(g_ref, p2_ref, aw_ref, av_ref, rw_ref, out_ref):
    agg = _gat_block(g_ref[...], aw_ref[...], av_ref[...])
    acc = (jnp.dot(agg, rw_ref[...], preferred_element_type=jnp.float32)
           + p2_ref[...])
    out_ref[...] = jnp.tanh(acc)


_HOP_SPEC = pl.BlockSpec((BB * NP, D2), lambda i: (i, 0))
_ROW128_SPEC = pl.BlockSpec((BB, D2), lambda i: (i, 0))
_ROW_SPEC = pl.BlockSpec((BB, D), lambda i: (i, 0))
_AW_SPEC = pl.BlockSpec((D2, D), lambda i: (0, 0))
_AV_SPEC = pl.BlockSpec((D, 1), lambda i: (0, 0))
_OUT = jax.ShapeDtypeStruct((B, D), jnp.float32)


def _tc_hop1(g, aw_h, av, rw_h):
    return pl.pallas_call(
        _tc_hop1_body, grid=(B // BB,),
        in_specs=[_HOP_SPEC, _AW_SPEC, _AV_SPEC, _AW_SPEC],
        out_specs=_ROW_SPEC, out_shape=_OUT,
    )(g, aw_h, av, rw_h)


def _tc_hop2(g, gt, p1, aw_h, av, rw_h, rw_t):
    return pl.pallas_call(
        _tc_hop2_body, grid=(B // BB,),
        in_specs=[_HOP_SPEC, _ROW128_SPEC, _ROW_SPEC, _AW_SPEC, _AV_SPEC,
                  _AW_SPEC, _AW_SPEC],
        out_specs=_ROW_SPEC, out_shape=_OUT,
    )(g, gt, p1, aw_h, av, rw_h, rw_t)


def _tc_hop3(g, p2, aw_h, av, rw_h):
    return pl.pallas_call(
        _tc_hop3_body, grid=(B // BB,),
        in_specs=[_HOP_SPEC, _ROW_SPEC, _AW_SPEC, _AV_SPEC, _AW_SPEC],
        out_specs=_ROW_SPEC, out_shape=_OUT,
    )(g, p2, aw_h, av, rw_h)


def kernel(target_ids, support_1st, support_2nd, support_3rd,
           user_emb, item_emb, att_w, att_v, refine_w):
    combo = jnp.concatenate([item_emb.T, user_emb.T], axis=0).T

    def pad_idx(s):
        return jnp.concatenate([s, s[:, :NP - N]], axis=1).reshape(
            NW, N_CHUNKS, CHUNK)

    i1idx = pad_idx(support_1st)
    uidx = pad_idx(support_2nd)
    i3idx = pad_idx(support_3rd)

    sc_hop = _make_sc_hop()
    g1 = sc_hop(combo, i1idx)
    g2, gt = _make_sc_hop_tgt()(combo, uidx, target_ids)
    g3 = sc_hop(combo, i3idx)

    z = jnp.zeros((D, D), jnp.float32)
    awi = jnp.concatenate([att_w, z], axis=0)
    awu = jnp.concatenate([z, att_w], axis=0)
    av = att_v.reshape(D, 1)
    rw1 = jnp.concatenate([refine_w[0:D], z], axis=0)
    rw2 = jnp.concatenate([z, refine_w[D:2 * D]], axis=0)
    rw3 = jnp.concatenate([refine_w[2 * D:3 * D], z], axis=0)
    rwt = jnp.concatenate([z, refine_w[3 * D:4 * D]], axis=0)

    p1 = _tc_hop1(g1, awi, av, rw1)
    p2 = _tc_hop2(g2, gt, p1, awu, av, rw2, rwt)
    return _tc_hop3(g3, p2, awi, av, rw3)


# speedup vs baseline: 1.0856x; 1.0338x over previous
import jax
import jax.numpy as jnp
from jax import lax
from jax.experimental import pallas as pl
from jax.experimental.pallas import tpu as pltpu
from jax.experimental.pallas import tpu_sc as plsc

B = 4096
N = 50
NP = 56
D = 64
D2 = 2 * D
NC = 2
NS = 16
NW = NC * NS

BW = B // NW
CHUNK = 128
HOP_ROWS = B * NP
PER_W = HOP_ROWS // NW
N_CHUNKS = PER_W // CHUNK


def _hop_loop(combo, idx_hbm, out_hbm, wid, idx_v, rows_a, rows_b, sem, wsa, wsb):
    pltpu.sync_copy(idx_hbm.at[wid], idx_v)
    base = wid * PER_W

    def pair(c2, carry):
        for par, (buf, ws) in enumerate(((rows_a, wsa), (rows_b, wsb))):
            c = c2 * 2 + par

            @pl.when(c2 > 0)
            def _():
                pltpu.make_async_copy(
                    out_hbm.at[pl.ds(0, CHUNK)], buf, ws).wait()

            pltpu.async_copy(combo.at[idx_v.at[c]], buf, sem).wait()
            pltpu.async_copy(buf, out_hbm.at[pl.ds(base + c * CHUNK, CHUNK)], ws)
        return carry

    lax.fori_loop(0, N_CHUNKS // 2, pair, 0)
    pltpu.make_async_copy(out_hbm.at[pl.ds(0, CHUNK)], rows_a, wsa).wait()
    pltpu.make_async_copy(out_hbm.at[pl.ds(0, CHUNK)], rows_b, wsb).wait()


def _sc_hop_body(combo, idx, g, idx_v, rows_a, rows_b, sem, wsa, wsb):
    wid = lax.axis_index("s") * NC + lax.axis_index("c")
    _hop_loop(combo, idx, g, wid, idx_v, rows_a, rows_b, sem, wsa, wsb)


def _sc_hop_tgt_body(combo, idx, tgt, g, gt, idx_v, tidx_v, rows_a, rows_b,
                     sem, wsa, wsb):
    wid = lax.axis_index("s") * NC + lax.axis_index("c")
    _hop_loop(combo, idx, g, wid, idx_v, rows_a, rows_b, sem, wsa, wsb)
    pltpu.sync_copy(tgt.at[pl.ds(wid * BW, BW)], tidx_v)
    pltpu.async_copy(combo.at[tidx_v], rows_a.at[pl.ds(0, BW)], sem).wait()
    pltpu.async_copy(rows_a.at[pl.ds(0, BW)], gt.at[pl.ds(wid * BW, BW)],
                     wsa).wait()


_MESH = dict(core_axis_name="c", subcore_axis_name="s")
_CP = pltpu.CompilerParams(use_tc_tiling_on_sc=True)


def _make_sc_hop():
    return pl.kernel(
        _sc_hop_body,
        out_type=jax.ShapeDtypeStruct((HOP_ROWS, D2), jnp.float32),
        mesh=plsc.VectorSubcoreMesh(**_MESH),
        scratch_types=[
            pltpu.VMEM((N_CHUNKS, CHUNK), jnp.int32),
            pltpu.VMEM((CHUNK, D2), jnp.float32),
            pltpu.VMEM((CHUNK, D2), jnp.float32),
            pltpu.SemaphoreType.DMA,
            pltpu.SemaphoreType.DMA,
            pltpu.SemaphoreType.DMA,
        ],
        compiler_params=_CP,
    )


def _make_sc_hop_tgt():
    return pl.kernel(
        _sc_hop_tgt_body,
        out_type=(
            jax.ShapeDtypeStruct((HOP_ROWS, D2), jnp.float32),
            jax.ShapeDtypeStruct((B, D2), jnp.float32),
        ),
        mesh=plsc.VectorSubcoreMesh(**_MESH),
        scratch_types=[
            pltpu.VMEM((N_CHUNKS, CHUNK), jnp.int32),
            pltpu.VMEM((BW,), jnp.int32),
            pltpu.VMEM((CHUNK, D2), jnp.float32),
            pltpu.VMEM((CHUNK, D2), jnp.float32),
            pltpu.SemaphoreType.DMA,
            pltpu.SemaphoreType.DMA,
            pltpu.SemaphoreType.DMA,
        ],
        compiler_params=_CP,
    )


BB = 128


def _gat_block(embf, aw_h, av):
    t = jnp.tanh(jnp.dot(embf, aw_h, preferred_element_type=jnp.float32))
    s = jnp.dot(t, av, preferred_element_type=jnp.float32)
    s3 = s.reshape(BB, NP, 1)
    nidx = lax.broadcasted_iota(jnp.int32, (BB, NP, 1), 1)
    s3 = jnp.where(nidx < N, s3, -1e30)
    m = jnp.max(s3, axis=1, keepdims=True)
    e = jnp.exp(s3 - m)
    alpha = e / jnp.sum(e, axis=1, keepdims=True)
    w = embf * alpha.reshape(BB * NP, 1)
    return jnp.sum(w.reshape(BB, NP, D2), axis=1)


def _tc_hop1_body(g_ref, aw_ref, av_ref, rw_ref, p_ref):
    agg = _gat_block(g_ref[...], aw_ref[...], av_ref[...])
    p_ref[...] = jnp.dot(agg, rw_ref[...], preferred_element_type=jnp.float32)


def _tc_hop2_body(g_ref, gt_ref, p1_ref, aw_ref, av_ref, rw_ref, rwt_ref, p_ref):
    agg = _gat_block(g_ref[...], aw_ref[...], av_ref[...])
    p_ref[...] = (jnp.dot(agg, rw_ref[...], preferred_element_type=jnp.float32)
                  + jnp.dot(gt_ref[...], rwt_ref[...],
                            preferred_element_type=jnp.float32)
                  + p1_ref[...])


def _tc_hop3_body(g_ref, p2_ref, aw_ref, av_ref, rw_ref, out_ref):
    agg = _gat_block(g_ref[...], aw_ref[...], av_ref[...])
    acc = (jnp.dot(agg, rw_ref[...], preferred_element_type=jnp.float32)
           + p2_ref[...])
    out_ref[...] = jnp.tanh(acc)


_HOP_SPEC = pl.BlockSpec((BB * NP, D2), lambda i: (i, 0))
_ROW128_SPEC = pl.BlockSpec((BB, D2), lambda i: (i, 0))
_ROW_SPEC = pl.BlockSpec((BB, D), lambda i: (i, 0))
_AW_SPEC = pl.BlockSpec((D2, D), lambda i: (0, 0))
_AV_SPEC = pl.BlockSpec((D, 1), lambda i: (0, 0))
_OUT = jax.ShapeDtypeStruct((B, D), jnp.float32)


def _tc_hop1(g, aw_h, av, rw_h):
    return pl.pallas_call(
        _tc_hop1_body, grid=(B // BB,),
        in_specs=[_HOP_SPEC, _AW_SPEC, _AV_SPEC, _AW_SPEC],
        out_specs=_ROW_SPEC, out_shape=_OUT,
    )(g, aw_h, av, rw_h)


def _tc_hop2(g, gt, p1, aw_h, av, rw_h, rw_t):
    return pl.pallas_call(
        _tc_hop2_body, grid=(B // BB,),
        in_specs=[_HOP_SPEC, _ROW128_SPEC, _ROW_SPEC, _AW_SPEC, _AV_SPEC,
                  _AW_SPEC, _AW_SPEC],
        out_specs=_ROW_SPEC, out_shape=_OUT,
    )(g, gt, p1, aw_h, av, rw_h, rw_t)


def _tc_hop3(g, p2, aw_h, av, rw_h):
    return pl.pallas_call(
        _tc_hop3_body, grid=(B // BB,),
        in_specs=[_HOP_SPEC, _ROW_SPEC, _AW_SPEC, _AV_SPEC, _AW_SPEC],
        out_specs=_ROW_SPEC, out_shape=_OUT,
    )(g, p2, aw_h, av, rw_h)


def kernel(target_ids, support_1st, support_2nd, support_3rd,
           user_emb, item_emb, att_w, att_v, refine_w):
    combo = jnp.concatenate([item_emb.T, user_emb.T], axis=0).T

    def pad_idx(s):
        return jnp.concatenate([s, s[:, :NP - N]], axis=1).reshape(
            NW, N_CHUNKS, CHUNK)

    i1idx = pad_idx(support_1st)
    uidx = pad_idx(support_2nd)
    i3idx = pad_idx(support_3rd)

    sc_hop = _make_sc_hop()
    g1 = sc_hop(combo, i1idx)
    g2, gt = _make_sc_hop_tgt()(combo, uidx, target_ids)
    g3 = sc_hop(combo, i3idx)

    z = jnp.zeros((D, D), jnp.float32)
    awi = jnp.concatenate([att_w, z], axis=0)
    awu = jnp.concatenate([z, att_w], axis=0)
    av = att_v.reshape(D, 1)
    rw1 = jnp.concatenate([refine_w[0:D], z], axis=0)
    rw2 = jnp.concatenate([z, refine_w[D:2 * D]], axis=0)
    rw3 = jnp.concatenate([refine_w[2 * D:3 * D], z], axis=0)
    rwt = jnp.concatenate([z, refine_w[3 * D:4 * D]], axis=0)

    p1 = _tc_hop1(g1, awi, av, rw1)
    p2 = _tc_hop2(g2, gt, p1, awu, av, rw2, rwt)
    return _tc_hop3(g3, p2, awi, av, rw3)
